# baseline (speedup 1.0000x reference)
import functools

import jax
import jax.numpy as jnp
from jax import lax
from jax.experimental import pallas as pl
from jax.experimental.pallas import tpu as pltpu
from jax.experimental.pallas import tpu_sc as plsc

_FDIM = 128
_BATCH = 16384
_D = _FDIM // 2

_NC = 2
_NS = 16
_NW = _NC * _NS
_BPW = _BATCH // _NW
_IDXC = 128
_NCHUNK = _BPW // _IDXC
_G = 16
_GPC = _IDXC // _G

_INVD = [float(1.0 / (10000.0 ** (d / (_D - 1)))) for d in range(_D)]

_TWO_OVER_PI = 0.6366197723675814
_MAGIC = 12582912.0
_DP1 = 1.5703125
_DP2 = 4.837512969970703e-4
_DP3 = 7.549789948768648e-8
_S1, _S2, _S3 = -1.6666654611e-1, 8.3321608736e-3, -1.9515295891e-4
_C1, _C2, _C3 = 4.166664568298827e-2, -1.388731625493765e-3, 2.443315711809948e-5


def _sincos16(x):
    xk = x * _TWO_OVER_PI
    kf0 = xk + _MAGIC
    kf = kf0 - _MAGIC
    q = kf.astype(jnp.int32) & 3
    r = x - kf * _DP1
    r = r - kf * _DP2
    r = r - kf * _DP3
    z = r * r
    s = ((_S3 * z + _S2) * z + _S1) * z * r + r
    c = ((_C3 * z + _C2) * z + _C1) * z * z - 0.5 * z + 1.0
    swap = (q & 1) != 0
    sb = jnp.where(swap, c, s)
    cb = jnp.where(swap, s, c)
    sin_v = jnp.where((q & 2) != 0, -sb, sb)
    cos_v = jnp.where(((q + 1) & 2) != 0, -cb, cb)
    return sin_v, cos_v


def _sc_fused(t_r, label_r, table):
    mesh = plsc.VectorSubcoreMesh(core_axis_name="c", subcore_axis_name="s")

    @functools.partial(
        pl.kernel,
        mesh=mesh,
        out_type=jax.ShapeDtypeStruct((_BATCH, _FDIM), jnp.float32),
        scratch_types=[
            pltpu.VMEM((_NCHUNK, _IDXC), jnp.int32),
            pltpu.VMEM((_BPW,), jnp.float32),
            pltpu.VMEM((_BPW, _FDIM), jnp.float32),
            [pltpu.SemaphoreType.DMA] * _NCHUNK,
            pltpu.SemaphoreType.DMA,
        ],
        compiler_params=pltpu.CompilerParams(needs_layout_passes=False),
    )
    def k(t_hbm, label_hbm, table_hbm, out_hbm,
          idx_v, t_v, rows_v, gsems, osem):
        wid = lax.axis_index("s") * _NC + lax.axis_index("c")
        base = wid * _BPW
        pltpu.sync_copy(label_hbm.at[wid], idx_v)
        pltpu.sync_copy(t_hbm.at[wid], t_v)
        gathers = [
            pltpu.async_copy(
                table_hbm.at[idx_v.at[j]],
                rows_v.at[pl.ds(j * _IDXC, _IDXC)],
                gsems[j],
            )
            for j in range(_NCHUNK)
        ]
        lane = lax.iota(jnp.int32, _G)
        out_copies = []
        for j in range(_NCHUNK):
            gathers[j].wait()

            def body(g, _, j=j):
                e0 = j * _IDXC + g * _G
                tvec = t_v[pl.ds(e0, _G)]
                row_idx = lane + e0
                for d in range(_D):
                    sin_v, cos_v = _sincos16(tvec * _INVD[d])
                    col_s = jnp.full((_G,), d, jnp.int32)
                    col_c = jnp.full((_G,), d + _D, jnp.int32)
                    plsc.addupdate_scatter(rows_v, [row_idx, col_s], sin_v)
                    plsc.addupdate_scatter(rows_v, [row_idx, col_c], cos_v)
                return 0

            lax.fori_loop(0, _GPC, body, 0)
            out_copies.append(
                pltpu.async_copy(
                    rows_v.at[pl.ds(j * _IDXC, _IDXC)],
                    out_hbm.at[pl.ds(base + j * _IDXC, _IDXC)],
                    osem,
                )
            )
        for c in out_copies:
            c.wait()

    return k(t_r, label_r, table)


def kernel(t, label, class_emb):
    label_r = label.astype(jnp.int32).reshape(_NW, _NCHUNK, _IDXC)
    t_r = t.reshape(_NW, _BPW)
    return _sc_fused(t_r, label_r, class_emb)


# speedup vs baseline: 1.4448x; 1.4448x over previous
import functools

import jax
import jax.numpy as jnp
from jax import lax
from jax.experimental import pallas as pl
from jax.experimental.pallas import tpu as pltpu
from jax.experimental.pallas import tpu_sc as plsc

_FDIM = 128
_BATCH = 16384
_D = _FDIM // 2

_NC = 2
_NS = 16
_NW = _NC * _NS
_BPW = _BATCH // _NW
_IDXC = 128
_NCHUNK = _BPW // _IDXC


def _sc_gather(label_r, table):
    mesh = plsc.VectorSubcoreMesh(core_axis_name="c", subcore_axis_name="s")

    @functools.partial(
        pl.kernel,
        mesh=mesh,
        out_type=jax.ShapeDtypeStruct((_BATCH, _FDIM), jnp.float32),
        scratch_types=[
            pltpu.VMEM((_NCHUNK, _IDXC), jnp.int32),
            pltpu.VMEM((_BPW, _FDIM), jnp.float32),
            [pltpu.SemaphoreType.DMA] * _NCHUNK,
            pltpu.SemaphoreType.DMA,
        ],
    )
    def k(label_hbm, table_hbm, out_hbm, idx_v, rows_v, gsems, osem):
        wid = lax.axis_index("s") * _NC + lax.axis_index("c")
        base = wid * _BPW
        pltpu.sync_copy(label_hbm.at[wid], idx_v)
        gathers = [
            pltpu.async_copy(
                table_hbm.at[idx_v.at[j]],
                rows_v.at[pl.ds(j * _IDXC, _IDXC)],
                gsems[j],
            )
            for j in range(_NCHUNK)
        ]
        out_copies = []
        for j in range(_NCHUNK):
            gathers[j].wait()
            out_copies.append(
                pltpu.async_copy(
                    rows_v.at[pl.ds(j * _IDXC, _IDXC)],
                    out_hbm.at[pl.ds(base + j * _IDXC, _IDXC)],
                    osem,
                )
            )
        for c in out_copies:
            c.wait()

    return k(label_r, table)


def _tc_body(t_ref, denom_ref, g_ref, o_ref):
    targ = t_ref[...] / denom_ref[...]
    emb = jnp.concatenate((jnp.sin(targ), jnp.cos(targ)), axis=1)
    o_ref[...] = emb + g_ref[...]


def _tc_sin_add(t2, denom2, g):
    bb = 2048
    return pl.pallas_call(
        _tc_body,
        out_shape=jax.ShapeDtypeStruct((_BATCH, _FDIM), jnp.float32),
        grid=(_BATCH // bb,),
        in_specs=[
            pl.BlockSpec((bb, 1), lambda i: (i, 0)),
            pl.BlockSpec((1, _D), lambda i: (0, 0)),
            pl.BlockSpec((bb, _FDIM), lambda i: (i, 0)),
        ],
        out_specs=pl.BlockSpec((bb, _FDIM), lambda i: (i, 0)),
    )(t2, denom2, g)


def kernel(t, label, class_emb):
    label_r = label.astype(jnp.int32).reshape(_NW, _NCHUNK, _IDXC)
    gathered = _sc_gather(label_r, class_emb)
    denom = 10000.0 ** (jnp.arange(_D, dtype=jnp.float32) / (_D - 1))
    return _tc_sin_add(t.reshape(_BATCH, 1), denom.reshape(1, _D), gathered)


# speedup vs baseline: 2.1162x; 1.4648x over previous
import functools

import jax
import jax.numpy as jnp
from jax import lax
from jax.experimental import pallas as pl
from jax.experimental.pallas import tpu as pltpu
from jax.experimental.pallas import tpu_sc as plsc

_FDIM = 128
_BATCH = 16384
_D = _FDIM // 2

_NC = 2
_NS = 16
_NW = _NC * _NS
_BPW = _BATCH // _NW
_IDXC = 128
_NCHUNK = _BPW // _IDXC

_MAGIC = 12582912.0
_INV2PI = 0.15915494309189535
_HI = 6.28125
_LO = 0.0019353071795864769
_S = (0.9999845867745937, -0.1666325820429799, 0.00831238293380817,
      -0.00019316182195923057, 2.17321006809601e-06)
_C = (0.9999994434180968, -0.499995580367214, 0.04166103157430418,
      -0.0013862743260457874, 2.425313775122201e-05,
      -2.2193694176886325e-07)


def _sc_gather(label_r, table):
    mesh = plsc.VectorSubcoreMesh(core_axis_name="c", subcore_axis_name="s")

    @functools.partial(
        pl.kernel,
        mesh=mesh,
        out_type=jax.ShapeDtypeStruct((_BATCH, _FDIM), jnp.float32),
        scratch_types=[
            pltpu.VMEM((_NCHUNK, _IDXC), jnp.int32),
            pltpu.VMEM((_BPW, _FDIM), jnp.float32),
            [pltpu.SemaphoreType.DMA] * _NCHUNK,
            pltpu.SemaphoreType.DMA,
        ],
    )
    def k(label_hbm, table_hbm, out_hbm, idx_v, rows_v, gsems, osem):
        wid = lax.axis_index("s") * _NC + lax.axis_index("c")
        base = wid * _BPW
        pltpu.sync_copy(label_hbm.at[wid], idx_v)
        gathers = [
            pltpu.async_copy(
                table_hbm.at[idx_v.at[j]],
                rows_v.at[pl.ds(j * _IDXC, _IDXC)],
                gsems[j],
            )
            for j in range(_NCHUNK)
        ]
        out_copies = []
        for j in range(_NCHUNK):
            gathers[j].wait()
            out_copies.append(
                pltpu.async_copy(
                    rows_v.at[pl.ds(j * _IDXC, _IDXC)],
                    out_hbm.at[pl.ds(base + j * _IDXC, _IDXC)],
                    osem,
                )
            )
        for c in out_copies:
            c.wait()

    return k(label_r, table)


def _tc_body(t_ref, invd_ref, g_ref, o_ref):
    x = t_ref[...] * invd_ref[...]
    kf = jnp.floor(x * _INV2PI + 0.5)
    r = (x - kf * _HI) - kf * _LO
    z = r * r
    sp = _S[4]
    for a in (_S[3], _S[2], _S[1], _S[0]):
        sp = sp * z + a
    sin_v = sp * r
    cp = _C[5]
    for a in (_C[4], _C[3], _C[2], _C[1], _C[0]):
        cp = cp * z + a
    emb = jnp.concatenate((sin_v, cp), axis=1)
    o_ref[...] = emb + g_ref[...]


def _tc_sin_add(t2, invd2, g):
    bb = 2048
    return pl.pallas_call(
        _tc_body,
        out_shape=jax.ShapeDtypeStruct((_BATCH, _FDIM), jnp.float32),
        grid=(_BATCH // bb,),
        in_specs=[
            pl.BlockSpec((bb, 1), lambda i: (i, 0)),
            pl.BlockSpec((1, _D), lambda i: (0, 0)),
            pl.BlockSpec((bb, _FDIM), lambda i: (i, 0)),
        ],
        out_specs=pl.BlockSpec((bb, _FDIM), lambda i: (i, 0)),
    )(t2, invd2, g)


def kernel(t, label, class_emb):
    label_r = label.astype(jnp.int32).reshape(_NW, _NCHUNK, _IDXC)
    gathered = _sc_gather(label_r, class_emb)
    denom = 10000.0 ** (jnp.arange(_D, dtype=jnp.float32) / (_D - 1))
    invd = (1.0 / denom).reshape(1, _D)
    return _tc_sin_add(t.reshape(_BATCH, 1), invd, gathered)
